# baseline (speedup 1.0000x reference)
import jax
import jax.numpy as jnp
from jax import lax
from jax.experimental import pallas as pl
from jax.experimental.pallas import tpu as pltpu
from jax.experimental.pallas import tpu_sc as plsc

B = 4096
L = 1000
E = 8

SC_B = 1536
NC = 2
NS = 16
NW = NC * NS
RPW = SC_B // NW
G = 16
NGRP = RPW // G
ES = 4

BB = 256
TC_B = B - SC_B


def _sc_oracle(labels_hbm, *refs):
    logits_hbm = refs[:E]
    out_hbm = refs[E]
    labels_v = refs[E + 1]
    x_v = refs[E + 2]
    row_v = refs[E + 3]
    sem = refs[E + 4]

    wid = lax.axis_index("s") * NC + lax.axis_index("c")
    base = wid * RPW
    pltpu.sync_copy(labels_hbm.at[pl.ds(base, RPW)], labels_v)

    lane = lax.iota(jnp.int32, G)
    esplat = [jnp.full((G,), s, jnp.int32) for s in range(ES)]
    ninf = jnp.full((G,), -jnp.inf, jnp.float32)

    def group_body(g, carry):
        r0 = base + g * G
        lab = labels_v[pl.ds(g * G, G)]

        first_correct = jnp.full((G,), E, jnp.int32)
        fallback = jnp.zeros((G,), jnp.int32)
        best_ll = ninf
        prev_cur = jnp.full((G,), -1, jnp.int32)
        for stage in range(E // ES):
            e0 = stage * ES
            for s in range(ES):
                pltpu.make_async_copy(
                    logits_hbm[e0 + s].at[pl.ds(r0, G)], x_v.at[s],
                    sem).start()
            for s in range(ES):
                pltpu.make_async_copy(
                    logits_hbm[e0 + s].at[pl.ds(r0, G)], x_v.at[s],
                    sem).wait()

            init = (ninf,) * ES + ((jnp.zeros((G,), jnp.int32)),) * ES

            @plsc.parallel_loop(0, L, unroll=8, carry=init)
            def col_loop(l, c):
                colv = jnp.zeros((G,), jnp.int32) + l
                ms = list(c[:ES])
                mis = list(c[ES:])
                for s in range(ES):
                    v = plsc.load_gather(x_v, [esplat[s], lane, colv])
                    gt = v > ms[s]
                    ms[s] = jnp.where(gt, v, ms[s])
                    mis[s] = jnp.where(gt, colv, mis[s])
                return tuple(ms) + tuple(mis)

            res = col_loop
            mis = res[ES:]

            for s in range(ES):
                e = e0 + s
                ll = plsc.load_gather(x_v, [esplat[s], lane, lab])
                correct = mis[s] == lab
                first_correct = jnp.where(
                    (first_correct == E) & correct, e, first_correct)
                take = ll > best_ll
                fallback = jnp.where(take, e, fallback)
                best_ll = jnp.where(take, ll, best_ll)

            cur = jnp.where(first_correct < E, first_correct, fallback)
            upd = cur != prev_cur
            prev_cur = cur
            idx0 = jnp.maximum(cur - e0, 0)

            @plsc.parallel_loop(0, L, unroll=8)
            def out_loop(l):
                colv = jnp.zeros((G,), jnp.int32) + l
                v = plsc.load_gather(x_v, [idx0, lane, colv])
                plsc.store_scatter(row_v, [lane, colv], v, mask=upd)

        pltpu.sync_copy(row_v, out_hbm.at[pl.ds(r0, G)])
        return carry

    lax.fori_loop(0, NGRP, group_body, 0)


_sc_mesh = plsc.VectorSubcoreMesh(
    core_axis_name="c", subcore_axis_name="s", num_cores=NC, num_subcores=NS)

_sc_call = pl.kernel(
    _sc_oracle,
    out_type=jax.ShapeDtypeStruct((SC_B, L), jnp.float32),
    mesh=_sc_mesh,
    scratch_types=[
        pltpu.VMEM((RPW,), jnp.int32),
        pltpu.VMEM((ES, G, L), jnp.float32),
        pltpu.VMEM((G, L), jnp.float32),
        pltpu.SemaphoreType.DMA,
    ],
    compiler_params=pltpu.CompilerParams(needs_layout_passes=False),
)


def _tc_oracle_block(labels_ref, *refs):
    logits_refs = refs[:E]
    out_ref = refs[E]
    labels = labels_ref[...]
    iota = jax.lax.broadcasted_iota(jnp.int32, (BB, L), 1)
    lab_eq = iota == labels
    pre = iota < labels
    ninf = jnp.float32(-jnp.inf)

    first_correct = jnp.full((BB, 1), E, dtype=jnp.int32)
    fallback = jnp.full((BB, 1), 0, dtype=jnp.int32)
    best_ll = jnp.full((BB, 1), ninf, dtype=jnp.float32)
    for e in range(E):
        x = logits_refs[e][...]
        m = jnp.max(x, axis=1, keepdims=True)
        ll = jnp.max(jnp.where(lab_eq, x, ninf), axis=1, keepdims=True)
        pm = jnp.max(jnp.where(pre, x, ninf), axis=1, keepdims=True)
        correct = (ll >= m) & (pm < m)
        first_correct = jnp.where(
            (first_correct == E) & correct, e, first_correct)
        take = ll > best_ll
        fallback = jnp.where(take, e, fallback)
        best_ll = jnp.where(take, ll, best_ll)
    best = jnp.where(first_correct < E, first_correct, fallback)

    out = logits_refs[0][...]
    for e in range(1, E):
        out = jnp.where(best == e, logits_refs[e][...], out)
    out_ref[...] = out


def _tc_call(labels2, *logits):
    off = SC_B // BB
    logit_spec = pl.BlockSpec((BB, L), lambda i: (i + off, 0))
    return pl.pallas_call(
        _tc_oracle_block,
        grid=(TC_B // BB,),
        in_specs=[pl.BlockSpec((BB, 1), lambda i: (i + off, 0))]
        + [logit_spec] * E,
        out_specs=pl.BlockSpec((BB, L), lambda i: (i, 0)),
        out_shape=jax.ShapeDtypeStruct((TC_B, L), jnp.float32),
        compiler_params=pltpu.CompilerParams(
            dimension_semantics=("arbitrary",),
        ),
    )(labels2, *logits)


@jax.jit
def kernel(labels, logits_0, logits_1, logits_2, logits_3, logits_4,
           logits_5, logits_6, logits_7):
    logits = (logits_0, logits_1, logits_2, logits_3, logits_4, logits_5,
              logits_6, logits_7)
    labels_i = labels.astype(jnp.int32)
    sc_out = _sc_call(labels_i, *logits)
    tc_out = _tc_call(labels_i.reshape(B, 1), *logits)
    return jnp.concatenate([sc_out, tc_out], axis=0)


# speedup vs baseline: 1.6282x; 1.6282x over previous
import jax
import jax.numpy as jnp
from jax import lax
from jax.experimental import pallas as pl
from jax.experimental.pallas import tpu as pltpu
from jax.experimental.pallas import tpu_sc as plsc

B = 4096
L = 1000
E = 8

SC_B = 512
NC = 2
NS = 16
NW = NC * NS
RPW = SC_B // NW
G = 16
NGRP = RPW // G
ES = 4

BB = 256
TC_B = B - SC_B


def _sc_oracle(labels_hbm, *refs):
    logits_hbm = refs[:E]
    out_hbm = refs[E]
    labels_v = refs[E + 1]
    x_v = refs[E + 2]
    row_v = refs[E + 3]
    sem = refs[E + 4]

    wid = lax.axis_index("s") * NC + lax.axis_index("c")
    base = wid * RPW
    pltpu.sync_copy(labels_hbm.at[pl.ds(base, RPW)], labels_v)

    lane = lax.iota(jnp.int32, G)
    esplat = [jnp.full((G,), s, jnp.int32) for s in range(ES)]
    ninf = jnp.full((G,), -jnp.inf, jnp.float32)

    def group_body(g, carry):
        r0 = base + g * G
        lab = labels_v[pl.ds(g * G, G)]

        first_correct = jnp.full((G,), E, jnp.int32)
        fallback = jnp.zeros((G,), jnp.int32)
        best_ll = ninf
        prev_cur = jnp.full((G,), -1, jnp.int32)
        for stage in range(E // ES):
            e0 = stage * ES
            for s in range(ES):
                pltpu.make_async_copy(
                    logits_hbm[e0 + s].at[pl.ds(r0, G)], x_v.at[s],
                    sem).start()
            for s in range(ES):
                pltpu.make_async_copy(
                    logits_hbm[e0 + s].at[pl.ds(r0, G)], x_v.at[s],
                    sem).wait()

            init = (ninf,) * ES + ((jnp.zeros((G,), jnp.int32)),) * ES

            @plsc.parallel_loop(0, L, unroll=8, carry=init)
            def col_loop(l, c):
                colv = jnp.zeros((G,), jnp.int32) + l
                ms = list(c[:ES])
                mis = list(c[ES:])
                for s in range(ES):
                    v = plsc.load_gather(x_v, [esplat[s], lane, colv])
                    gt = v > ms[s]
                    ms[s] = jnp.where(gt, v, ms[s])
                    mis[s] = jnp.where(gt, colv, mis[s])
                return tuple(ms) + tuple(mis)

            res = col_loop
            mis = res[ES:]

            for s in range(ES):
                e = e0 + s
                ll = plsc.load_gather(x_v, [esplat[s], lane, lab])
                correct = mis[s] == lab
                first_correct = jnp.where(
                    (first_correct == E) & correct, e, first_correct)
                take = ll > best_ll
                fallback = jnp.where(take, e, fallback)
                best_ll = jnp.where(take, ll, best_ll)

            cur = jnp.where(first_correct < E, first_correct, fallback)
            upd = cur != prev_cur
            prev_cur = cur
            idx0 = jnp.maximum(cur - e0, 0)

            @plsc.parallel_loop(0, L, unroll=8)
            def out_loop(l):
                colv = jnp.zeros((G,), jnp.int32) + l
                v = plsc.load_gather(x_v, [idx0, lane, colv])
                plsc.store_scatter(row_v, [lane, colv], v, mask=upd)

        pltpu.sync_copy(row_v, out_hbm.at[pl.ds(r0, G)])
        return carry

    lax.fori_loop(0, NGRP, group_body, 0)


_sc_mesh = plsc.VectorSubcoreMesh(
    core_axis_name="c", subcore_axis_name="s", num_cores=NC, num_subcores=NS)

_sc_call = pl.kernel(
    _sc_oracle,
    out_type=jax.ShapeDtypeStruct((SC_B, L), jnp.float32),
    mesh=_sc_mesh,
    scratch_types=[
        pltpu.VMEM((RPW,), jnp.int32),
        pltpu.VMEM((ES, G, L), jnp.float32),
        pltpu.VMEM((G, L), jnp.float32),
        pltpu.SemaphoreType.DMA,
    ],
    compiler_params=pltpu.CompilerParams(needs_layout_passes=False),
)


def _tc_oracle_block(labels_ref, *refs):
    logits_refs = refs[:E]
    out_ref = refs[E]
    labels = labels_ref[...]
    iota = jax.lax.broadcasted_iota(jnp.int32, (BB, L), 1)
    lab_eq = iota == labels
    pre = iota < labels
    ninf = jnp.float32(-jnp.inf)

    first_correct = jnp.full((BB, 1), E, dtype=jnp.int32)
    fallback = jnp.full((BB, 1), 0, dtype=jnp.int32)
    best_ll = jnp.full((BB, 1), ninf, dtype=jnp.float32)
    for e in range(E):
        x = logits_refs[e][...]
        m = jnp.max(x, axis=1, keepdims=True)
        ll = jnp.max(jnp.where(lab_eq, x, ninf), axis=1, keepdims=True)
        pm = jnp.max(jnp.where(pre, x, ninf), axis=1, keepdims=True)
        correct = (ll >= m) & (pm < m)
        first_correct = jnp.where(
            (first_correct == E) & correct, e, first_correct)
        take = ll > best_ll
        fallback = jnp.where(take, e, fallback)
        best_ll = jnp.where(take, ll, best_ll)
    best = jnp.where(first_correct < E, first_correct, fallback)

    out = logits_refs[0][...]
    for e in range(1, E):
        out = jnp.where(best == e, logits_refs[e][...], out)
    out_ref[...] = out


def _tc_call(labels2, *logits):
    off = SC_B // BB
    logit_spec = pl.BlockSpec((BB, L), lambda i: (i + off, 0))
    return pl.pallas_call(
        _tc_oracle_block,
        grid=(TC_B // BB,),
        in_specs=[pl.BlockSpec((BB, 1), lambda i: (i + off, 0))]
        + [logit_spec] * E,
        out_specs=pl.BlockSpec((BB, L), lambda i: (i, 0)),
        out_shape=jax.ShapeDtypeStruct((TC_B, L), jnp.float32),
        compiler_params=pltpu.CompilerParams(
            dimension_semantics=("arbitrary",),
        ),
    )(labels2, *logits)


@jax.jit
def kernel(labels, logits_0, logits_1, logits_2, logits_3, logits_4,
           logits_5, logits_6, logits_7):
    logits = (logits_0, logits_1, logits_2, logits_3, logits_4, logits_5,
              logits_6, logits_7)
    labels_i = labels.astype(jnp.int32)
    sc_out = _sc_call(labels_i, *logits)
    tc_out = _tc_call(labels_i.reshape(B, 1), *logits)
    return jnp.concatenate([sc_out, tc_out], axis=0)
